# baseline (speedup 1.0000x reference)
import functools

import jax
import jax.numpy as jnp
from jax import lax
from jax.experimental import pallas as pl
from jax.experimental.pallas import tpu as pltpu
from jax.experimental.pallas import tpu_sc as plsc

_B = 16384
_H = 128
_NC = 2
_NS = 16
_NW = _NC * _NS
_NSPLIT = 2
_HB = _B // _NSPLIT
_RPW = _HB // _NW
_CHUNK = 128
_NCHUNK = _RPW // _CHUNK
_NWORK = 3 * _NCHUNK
_DEPTH = _NWORK


def _sc_gather_body(spas_t, wl_t, loc_t, spas_i, wl_i, loc_i,
                    out_s, out_w, out_l, idx_v, bufs, gsem, ssem):
    wid = lax.axis_index("s") * _NC + lax.axis_index("c")
    base = wid * _RPW
    tables = (spas_t, wl_t, loc_t)
    idxs = (spas_i, wl_i, loc_i)
    outs = (out_s, out_w, out_l)

    for t in range(3):
        pltpu.sync_copy(idxs[t].at[pl.ds(wid * _NCHUNK, _NCHUNK)], idx_v.at[t])

    work = [(t, j) for t in range(3) for j in range(_NCHUNK)]
    n = len(work)

    def gather(k):
        t, j = work[k]
        return pltpu.async_copy(tables[t].at[idx_v.at[t].at[j]],
                                bufs.at[k % _DEPTH], gsem)

    def store(k):
        t, j = work[k]
        return pltpu.async_copy(bufs.at[k % _DEPTH],
                                outs[t].at[pl.ds(base + j * _CHUNK, _CHUNK)],
                                ssem)

    gs = [gather(k) for k in range(min(_DEPTH, n))]
    stores = [None] * n
    for k in range(n):
        gs[k].wait()
        stores[k] = store(k)
        if k + _DEPTH < n:
            stores[k].wait()
            gs.append(gather(k + _DEPTH))
    for k in range(max(0, n - _DEPTH), n):
        stores[k].wait()


_sc_gather = functools.partial(
    pl.kernel,
    out_type=(jax.ShapeDtypeStruct((_HB, _H), jnp.float32),) * 3,
    mesh=plsc.VectorSubcoreMesh(core_axis_name="c", subcore_axis_name="s",
                                num_cores=_NC, num_subcores=_NS),
    scratch_types=[
        pltpu.VMEM((3, _NCHUNK, _CHUNK), jnp.int32),
        pltpu.VMEM((_DEPTH, _CHUNK, _H), jnp.float32),
        pltpu.SemaphoreType.DMA,
        pltpu.SemaphoreType.DMA,
    ],
)(_sc_gather_body)


_BS = 4096
_GRID = _HB // _BS

_DN0 = (((0,), (0,)), ((), ()))


def _tc_body(o_prev_ref, xt1_ref, es_ref, ew_ref, el_ref,
             w1b_ref, w2_ref, b2_ref, w3_ref, b3_ref, o_ref):
    del o_prev_ref
    bf = jnp.bfloat16
    f32 = jnp.float32
    w3 = w3_ref[...].astype(bf)
    w3d = w3[3 * _H:4 * _H, :]
    h1_t = jnp.maximum(
        lax.dot_general(w1b_ref[...].astype(bf), xt1_ref[...].astype(bf),
                        _DN0, preferred_element_type=f32), 0.0)
    w4 = jnp.dot(w2_ref[...].astype(bf), w3d,
                 preferred_element_type=f32).astype(bf)
    b34 = jnp.dot(b2_ref[...].astype(bf), w3d,
                  preferred_element_type=f32) + b3_ref[...]
    acc = lax.dot_general(h1_t.astype(bf), w4, _DN0,
                          preferred_element_type=f32)
    acc += jnp.dot(es_ref[...].astype(bf), w3[0:_H, :],
                   preferred_element_type=f32)
    acc += jnp.dot(ew_ref[...].astype(bf), w3[_H:2 * _H, :],
                   preferred_element_type=f32)
    acc += jnp.dot(el_ref[...].astype(bf), w3[2 * _H:3 * _H, :],
                   preferred_element_type=f32)
    o_ref[...] = jnp.maximum(acc + b34, 0.0)


def _tc_project(split, o_prev, xt1, es, ew, el, W1b, W2, b2, W3, b3):
    off = split * _GRID
    batch = pl.BlockSpec((_BS, _H), lambda i: (i, 0))
    full = lambda s: pl.BlockSpec(s, lambda i: (0, 0))
    return pl.pallas_call(
        _tc_body,
        grid=(_GRID,),
        in_specs=[pl.BlockSpec(memory_space=pl.ANY),
                  pl.BlockSpec((3, _BS), lambda i: (0, i + off)),
                  batch, batch, batch,
                  full((3, _H)), full((_H, _H)), full((1, _H)),
                  full((4 * _H, _H)), full((1, _H))],
        out_specs=pl.BlockSpec((_BS, _H), lambda i: (i + off, 0)),
        out_shape=jax.ShapeDtypeStruct((_B, _H), jnp.float32),
        input_output_aliases={0: 0},
    )(o_prev, xt1, es, ew, el, W1b, W2, b2, W3, b3)


def kernel(spas_item_id, wl_id, wf_loc_id, wf_loc_x, wf_loc_y,
           spas_table, wl_table, loc_table, W1, b1, W2, b2, W3, b3):
    si = spas_item_id.astype(jnp.int32).reshape(_B // _CHUNK, _CHUNK)
    wi = wl_id.astype(jnp.int32).reshape(_B // _CHUNK, _CHUNK)
    li = wf_loc_id.astype(jnp.int32).reshape(_B // _CHUNK, _CHUNK)
    xt1 = jnp.stack([wf_loc_x, wf_loc_y, jnp.ones_like(wf_loc_x)])
    W1b = jnp.concatenate([W1, b1[None, :]])
    b2r, b3r = b2[None, :], b3[None, :]

    nrow = _HB // _CHUNK
    gathered = [
        _sc_gather(spas_table, wl_table, loc_table,
                   si[s * nrow:(s + 1) * nrow],
                   wi[s * nrow:(s + 1) * nrow],
                   li[s * nrow:(s + 1) * nrow])
        for s in range(_NSPLIT)
    ]
    out = jnp.zeros((_B, _H), jnp.float32)
    for s in range(_NSPLIT):
        es, ew, el = gathered[s]
        out = _tc_project(s, out, xt1, es, ew, el, W1b, W2, b2r, W3, b3r)
    return out


# speedup vs baseline: 1.0609x; 1.0609x over previous
import functools

import jax
import jax.numpy as jnp
from jax import lax
from jax.experimental import pallas as pl
from jax.experimental.pallas import tpu as pltpu
from jax.experimental.pallas import tpu_sc as plsc

_B = 16384
_H = 128
_NC = 2
_NS = 16
_NW = _NC * _NS
_NSPLIT = 2
_HB = _B // _NSPLIT
_RPW = _HB // _NW
_CHUNK = 128
_NCHUNK = _RPW // _CHUNK
_NTAB = 2
_NWORK = _NTAB * _NCHUNK
_DEPTH = _NWORK
_NWL = 1024


def _sc_gather_body(spas_t, loc_t, spas_i, loc_i,
                    out_s, out_l, idx_v, bufs, gsem, ssem):
    wid = lax.axis_index("s") * _NC + lax.axis_index("c")
    base = wid * _RPW
    tables = (spas_t, loc_t)
    idxs = (spas_i, loc_i)
    outs = (out_s, out_l)

    for t in range(_NTAB):
        pltpu.sync_copy(idxs[t].at[pl.ds(wid * _NCHUNK, _NCHUNK)], idx_v.at[t])

    work = [(t, j) for t in range(_NTAB) for j in range(_NCHUNK)]
    n = len(work)

    def gather(k):
        t, j = work[k]
        return pltpu.async_copy(tables[t].at[idx_v.at[t].at[j]],
                                bufs.at[k % _DEPTH], gsem)

    def store(k):
        t, j = work[k]
        return pltpu.async_copy(bufs.at[k % _DEPTH],
                                outs[t].at[pl.ds(base + j * _CHUNK, _CHUNK)],
                                ssem)

    gs = [gather(k) for k in range(min(_DEPTH, n))]
    stores = [None] * n
    for k in range(n):
        gs[k].wait()
        stores[k] = store(k)
        if k + _DEPTH < n:
            stores[k].wait()
            gs.append(gather(k + _DEPTH))
    for k in range(max(0, n - _DEPTH), n):
        stores[k].wait()


_sc_gather = functools.partial(
    pl.kernel,
    out_type=(jax.ShapeDtypeStruct((_HB, _H), jnp.float32),) * _NTAB,
    mesh=plsc.VectorSubcoreMesh(core_axis_name="c", subcore_axis_name="s",
                                num_cores=_NC, num_subcores=_NS),
    scratch_types=[
        pltpu.VMEM((_NTAB, _NCHUNK, _CHUNK), jnp.int32),
        pltpu.VMEM((_DEPTH, _CHUNK, _H), jnp.float32),
        pltpu.SemaphoreType.DMA,
        pltpu.SemaphoreType.DMA,
    ],
)(_sc_gather_body)


_BS = 4096
_GRID = _HB // _BS

_DN0 = (((0,), (0,)), ((), ()))


def _tc_body(o_prev_ref, xt1_ref, es_ref, el_ref, wli_ref, wlt_ref,
             w1b_ref, w2_ref, b2_ref, w3_ref, b3_ref, o_ref):
    del o_prev_ref
    bf = jnp.bfloat16
    f32 = jnp.float32
    w3 = w3_ref[...].astype(bf)
    w3d = w3[3 * _H:4 * _H, :]
    h1_t = jnp.maximum(
        lax.dot_general(w1b_ref[...].astype(bf), xt1_ref[...].astype(bf),
                        _DN0, preferred_element_type=f32), 0.0)
    w4 = jnp.dot(w2_ref[...].astype(bf), w3d,
                 preferred_element_type=f32).astype(bf)
    b34 = jnp.dot(b2_ref[...].astype(bf), w3d,
                  preferred_element_type=f32) + b3_ref[...]
    acc = lax.dot_general(h1_t.astype(bf), w4, _DN0,
                          preferred_element_type=f32)
    wlw3b = jnp.dot(wlt_ref[...].astype(bf), w3[_H:2 * _H, :],
                    preferred_element_type=f32).astype(bf)
    oh = (lax.broadcasted_iota(jnp.int32, (_NWL, _BS), 0)
          == wli_ref[...]).astype(bf)
    acc += lax.dot_general(oh, wlw3b, _DN0, preferred_element_type=f32)
    acc += jnp.dot(es_ref[...].astype(bf), w3[0:_H, :],
                   preferred_element_type=f32)
    acc += jnp.dot(el_ref[...].astype(bf), w3[2 * _H:3 * _H, :],
                   preferred_element_type=f32)
    o_ref[...] = jnp.maximum(acc + b34, 0.0)


def _tc_project(split, o_prev, xt1, es, el, wli, wlt, W1b, W2, b2, W3, b3):
    off = split * _GRID
    batch = pl.BlockSpec((_BS, _H), lambda i: (i, 0))
    full = lambda s: pl.BlockSpec(s, lambda i: (0, 0))
    return pl.pallas_call(
        _tc_body,
        grid=(_GRID,),
        in_specs=[pl.BlockSpec(memory_space=pl.ANY),
                  pl.BlockSpec((3, _BS), lambda i: (0, i + off)),
                  batch, batch,
                  pl.BlockSpec((1, _BS), lambda i: (0, i + off)),
                  full((_NWL, _H)),
                  full((3, _H)), full((_H, _H)), full((1, _H)),
                  full((4 * _H, _H)), full((1, _H))],
        out_specs=pl.BlockSpec((_BS, _H), lambda i: (i + off, 0)),
        out_shape=jax.ShapeDtypeStruct((_B, _H), jnp.float32),
        input_output_aliases={0: 0},
    )(o_prev, xt1, es, el, wli, wlt, W1b, W2, b2, W3, b3)


def kernel(spas_item_id, wl_id, wf_loc_id, wf_loc_x, wf_loc_y,
           spas_table, wl_table, loc_table, W1, b1, W2, b2, W3, b3):
    si = spas_item_id.astype(jnp.int32).reshape(_B // _CHUNK, _CHUNK)
    li = wf_loc_id.astype(jnp.int32).reshape(_B // _CHUNK, _CHUNK)
    wli = wl_id.astype(jnp.int32)[None, :]
    wlt = jnp.pad(wl_table, ((0, _NWL - wl_table.shape[0]), (0, 0)))
    xt1 = jnp.stack([wf_loc_x, wf_loc_y, jnp.ones_like(wf_loc_x)])
    W1b = jnp.concatenate([W1, b1[None, :]])
    b2r, b3r = b2[None, :], b3[None, :]

    nrow = _HB // _CHUNK
    gathered = [
        _sc_gather(spas_table, loc_table,
                   si[s * nrow:(s + 1) * nrow],
                   li[s * nrow:(s + 1) * nrow])
        for s in range(_NSPLIT)
    ]
    out = jnp.zeros((_B, _H), jnp.float32)
    for s in range(_NSPLIT):
        es, el = gathered[s]
        out = _tc_project(s, out, xt1, es, el, wli, wlt, W1b, W2, b2r, W3, b3r)
    return out
